# Initial kernel scaffold; baseline (speedup 1.0000x reference)
import functools

import jax
import jax.numpy as jnp
from jax import lax
from jax.experimental import pallas as pl
from jax.experimental.pallas import tpu as pltpu
from jax.experimental.pallas import tpu_sc as plsc

_B, _DIM, _N = 8, 64, 8192
_T, _K = 512, 4
_NB = 512
_NBLK = _N // _NB
_Q = _B * _N
_NEG = float("-inf")
_SUB = 512


def _topk_body(tok_ref, src_ref, idx_ref):
    tok = tok_ref[0]
    inv = lax.rsqrt(jnp.sum(tok * tok, axis=0, keepdims=True))
    tokn = tok * inv
    s = src_ref[0]
    sc = lax.dot_general(tokn, s, (((0,), (0,)), ((), ())),
                         preferred_element_type=jnp.float32)
    row = lax.broadcasted_iota(jnp.int32, (_T, _NB), 0)
    for k in range(_K):
        m = jnp.max(sc, axis=0, keepdims=True)
        cand = jnp.where(sc == m, row, _T)
        t_star = jnp.min(cand, axis=0, keepdims=True)
        idx_ref[k:k + 1, :] = t_star
        sc = jnp.where(row == t_star, _NEG, sc)


_tc_topk = pl.pallas_call(
    _topk_body,
    grid=(_B, _NBLK),
    in_specs=[
        pl.BlockSpec((1, _DIM, _T), lambda b, j: (0, 0, 0)),
        pl.BlockSpec((1, _DIM, _NB), lambda b, j: (b, 0, j)),
    ],
    out_specs=pl.BlockSpec((_K, _NB), lambda b, j: (0, b * _NBLK + j)),
    out_shape=jax.ShapeDtypeStruct((_K, _Q), jnp.int32),
)


@functools.lru_cache(maxsize=1)
def _make_sc_gather():
    info = plsc.get_sparse_core_info()
    nc, ns = info.num_cores, info.num_subcores
    nw = nc * ns
    qpt = _Q // nw
    nsub = qpt // _SUB
    mesh = plsc.VectorSubcoreMesh(core_axis_name="c", subcore_axis_name="s")

    @functools.partial(
        pl.kernel,
        mesh=mesh,
        out_type=jax.ShapeDtypeStruct((_B, _DIM, _N), jnp.float32),
        scratch_types=[
            pltpu.VMEM((_DIM * _T,), jnp.float32),
            pltpu.VMEM((_K, _SUB), jnp.int32),
            pltpu.VMEM((_DIM, _SUB), jnp.float32),
        ],
    )
    def sc_gather(tok_hbm, idx_hbm, out_hbm, tab, idxv, outv):
        wid = lax.axis_index("s") * nc + lax.axis_index("c")
        pltpu.sync_copy(tok_hbm, tab)
        for c in range(nsub):
            q0 = wid * qpt + c * _SUB
            b = q0 // _N
            n0 = q0 - b * _N
            pltpu.sync_copy(idx_hbm.at[:, pl.ds(q0, _SUB)], idxv)

            def body(i, carry):
                s16 = pl.ds(i * 16, 16)
                iv = [idxv[k, s16] for k in range(_K)]
                for d in range(_DIM):
                    off = jnp.full((16,), d * _T, dtype=jnp.int32)
                    acc = plsc.load_gather(tab, [iv[0] + off])
                    for k in range(1, _K):
                        acc = acc + plsc.load_gather(tab, [iv[k] + off])
                    outv[d, s16] = acc * 0.25
                return carry

            lax.fori_loop(0, _SUB // 16, body, 0)
            pltpu.sync_copy(outv, out_hbm.at[b, :, pl.ds(n0, _SUB)])

    return sc_gather


def kernel(source, tokens):
    idx = _tc_topk(tokens, source)
    sc_gather = _make_sc_gather()
    return sc_gather(tokens.reshape(_DIM * _T), idx)


# speedup vs baseline: 22.7533x; 22.7533x over previous
import functools

import jax
import jax.numpy as jnp
from jax import lax
from jax.experimental import pallas as pl
from jax.experimental.pallas import tpu as pltpu
from jax.experimental.pallas import tpu_sc as plsc

_B, _DIM, _N = 8, 64, 8192
_T, _K = 512, 4
_NB = 512
_NBLK = _N // _NB
_Q = _B * _N
_NEG = float("-inf")
_SUB = 512


def _topk_body(tok_ref, src_ref, idx_ref):
    tok = tok_ref[0]
    tokn = tok / jnp.sqrt(jnp.sum(tok * tok, axis=0, keepdims=True))
    s = src_ref[0]
    sn = s / jnp.sqrt(jnp.sum(s * s, axis=0, keepdims=True))
    sc = lax.dot_general(tokn.astype(jnp.bfloat16), sn.astype(jnp.bfloat16),
                         (((0,), (0,)), ((), ())),
                         preferred_element_type=jnp.float32)
    row = lax.broadcasted_iota(jnp.int32, (_T, _NB), 0)
    for k in range(_K):
        m = jnp.max(sc, axis=0, keepdims=True)
        cand = jnp.where(sc == m, row, _T)
        t_star = jnp.min(cand, axis=0, keepdims=True)
        idx_ref[k:k + 1, :] = t_star
        sc = jnp.where(row == t_star, _NEG, sc)


_tc_topk = pl.pallas_call(
    _topk_body,
    grid=(_B, _NBLK),
    in_specs=[
        pl.BlockSpec((1, _DIM, _T), lambda b, j: (0, 0, 0)),
        pl.BlockSpec((1, _DIM, _NB), lambda b, j: (b, 0, j)),
    ],
    out_specs=pl.BlockSpec((_K, _NB), lambda b, j: (0, b * _NBLK + j)),
    out_shape=jax.ShapeDtypeStruct((_K, _Q), jnp.int32),
)


@functools.lru_cache(maxsize=1)
def _make_sc_gather():
    info = plsc.get_sparse_core_info()
    nc, ns = info.num_cores, info.num_subcores
    nw = nc * ns
    qpt = _Q // nw
    nsub = qpt // _SUB
    mesh = plsc.VectorSubcoreMesh(core_axis_name="c", subcore_axis_name="s")

    @functools.partial(
        pl.kernel,
        mesh=mesh,
        out_type=jax.ShapeDtypeStruct((_B, _DIM, _N), jnp.float32),
        scratch_types=[
            pltpu.VMEM((_DIM * _T,), jnp.float32),
            pltpu.VMEM((_K, _SUB), jnp.int32),
            pltpu.VMEM((_DIM, _SUB), jnp.float32),
        ],
        compiler_params=pltpu.CompilerParams(needs_layout_passes=False),
    )
    def sc_gather(tok_hbm, idx_hbm, out_hbm, tab, idxv, outv):
        wid = lax.axis_index("s") * nc + lax.axis_index("c")
        pltpu.sync_copy(tok_hbm, tab)
        for c in range(nsub):
            q0 = wid * qpt + c * _SUB
            b = q0 // _N
            n0 = q0 - b * _N
            pltpu.sync_copy(idx_hbm.at[:, pl.ds(q0, _SUB)], idxv)

            def body(i, carry):
                s16 = pl.ds(i * 16, 16)
                iv = [idxv[k, s16] for k in range(_K)]
                for d in range(_DIM):
                    off = jnp.full((16,), d * _T, dtype=jnp.int32)
                    acc = plsc.load_gather(tab, [iv[0] + off])
                    for k in range(1, _K):
                        acc = acc + plsc.load_gather(tab, [iv[k] + off])
                    outv[d, s16] = acc * 0.25
                return carry

            lax.fori_loop(0, _SUB // 16, body, 0)
            pltpu.sync_copy(outv, out_hbm.at[b, :, pl.ds(n0, _SUB)])

    return sc_gather


def kernel(source, tokens):
    idx = _tc_topk(tokens, source)
    sc_gather = _make_sc_gather()
    return sc_gather(tokens.reshape(_DIM * _T), idx)


# speedup vs baseline: 29.5296x; 1.2978x over previous
import functools

import jax
import jax.numpy as jnp
from jax import lax
from jax.experimental import pallas as pl
from jax.experimental.pallas import tpu as pltpu
from jax.experimental.pallas import tpu_sc as plsc

_B, _DIM, _N = 8, 64, 8192
_T, _K = 512, 4
_NB = 1024
_NBLK = _N // _NB
_Q = _B * _N
_NEG = float("-inf")
_SUB = 512


def _topk_body(tok_ref, src_ref, idx_ref):
    tok = tok_ref[0]
    tokn = tok / jnp.sqrt(jnp.sum(tok * tok, axis=0, keepdims=True))
    s = src_ref[0]
    sn = s / jnp.sqrt(jnp.sum(s * s, axis=0, keepdims=True))
    sc = lax.dot_general(tokn.astype(jnp.bfloat16), sn.astype(jnp.bfloat16),
                         (((0,), (0,)), ((), ())),
                         preferred_element_type=jnp.float32)
    rowf = lax.broadcasted_iota(jnp.int32, (_T, _NB), 0).astype(jnp.float32)
    for k in range(_K):
        m = jnp.max(sc, axis=0, keepdims=True)
        cand = jnp.where(sc == m, rowf, 1e9)
        t_star = jnp.min(cand, axis=0, keepdims=True)
        idx_ref[k:k + 1, :] = t_star.astype(jnp.int32)
        sc = jnp.where(cand == t_star, _NEG, sc)


_tc_topk = pl.pallas_call(
    _topk_body,
    grid=(_B, _NBLK),
    in_specs=[
        pl.BlockSpec((1, _DIM, _T), lambda b, j: (0, 0, 0)),
        pl.BlockSpec((1, _DIM, _NB), lambda b, j: (b, 0, j)),
    ],
    out_specs=pl.BlockSpec((_K, _NB), lambda b, j: (0, b * _NBLK + j)),
    out_shape=jax.ShapeDtypeStruct((_K, _Q), jnp.int32),
)


@functools.lru_cache(maxsize=1)
def _make_sc_gather():
    info = plsc.get_sparse_core_info()
    nc, ns = info.num_cores, info.num_subcores
    nw = nc * ns
    qpt = _Q // nw
    nsub = qpt // _SUB
    mesh = plsc.VectorSubcoreMesh(core_axis_name="c", subcore_axis_name="s")

    @functools.partial(
        pl.kernel,
        mesh=mesh,
        out_type=jax.ShapeDtypeStruct((_B, _DIM, _N), jnp.float32),
        scratch_types=[
            pltpu.VMEM((_DIM * _T,), jnp.float32),
            pltpu.VMEM((_K, _SUB), jnp.int32),
            pltpu.VMEM((_DIM, _SUB), jnp.float32),
        ],
        compiler_params=pltpu.CompilerParams(needs_layout_passes=False),
    )
    def sc_gather(tok_hbm, idx_hbm, out_hbm, tab, idxv, outv):
        wid = lax.axis_index("s") * nc + lax.axis_index("c")
        pltpu.sync_copy(tok_hbm, tab)
        for c in range(nsub):
            q0 = wid * qpt + c * _SUB
            b = q0 // _N
            n0 = q0 - b * _N
            pltpu.sync_copy(idx_hbm.at[:, pl.ds(q0, _SUB)], idxv)

            @plsc.parallel_loop(0, _SUB // 16, unroll=2)
            def _body(i):
                s16 = pl.ds(i * 16, 16)
                iv = [idxv[k, s16] for k in range(_K)]
                for d in range(_DIM):
                    off = jnp.full((16,), d * _T, dtype=jnp.int32)
                    g = [plsc.load_gather(tab, [iv[k] + off])
                         for k in range(_K)]
                    outv[d, s16] = ((g[0] + g[1]) + (g[2] + g[3])) * 0.25
            pltpu.sync_copy(outv, out_hbm.at[b, :, pl.ds(n0, _SUB)])

    return sc_gather


def kernel(source, tokens):
    idx = _tc_topk(tokens, source)
    sc_gather = _make_sc_gather()
    return sc_gather(tokens.reshape(_DIM * _T), idx)
